# Initial kernel scaffold; baseline (speedup 1.0000x reference)
import jax
import jax.numpy as jnp
from jax.experimental import pallas as pl


def kernel(timestamp, inst_ids, tick_price, cur_price):
    raise NotImplementedError("write your pallas kernel here")


# speedup vs baseline: 18.5833x; 18.5833x over previous
import functools

import jax
import jax.numpy as jnp
from jax import lax
from jax.experimental import pallas as pl
from jax.experimental.pallas import tpu as pltpu
from jax.experimental.pallas import tpu_sc as plsc

NUM_INST = 50000
N_TICKS = 3200000

NC = 2
NS = 16
L = 16
NW = NC * NS

C = 1568
NIDS = C * NW
BLK = 8192
SEARCH_STEPS = 22

_i32 = jnp.int32
_f32 = jnp.float32


def _sc_body(ids_hbm, prc_hbm, cur_hbm, out_hbm,
             acc_hi, acc_lo, ids_buf, prc_buf, cur_buf, sig_buf, probe_buf,
             sem_probe, sem_blk):
    lane = jnp.arange(L, dtype=_i32)
    w = lax.axis_index("s") * NC + lax.axis_index("c")
    base = w * C

    neg_inf = jnp.full((L,), -jnp.inf, dtype=_f32)
    pos_inf = jnp.full((L,), jnp.inf, dtype=_f32)

    def init_body(j, carry):
        off = pl.multiple_of(j * L, L)
        acc_hi[pl.ds(off, L)] = neg_inf
        acc_lo[pl.ds(off, L)] = pos_inf
        return carry

    lax.fori_loop(0, (L * C) // L, init_body, 0)

    pltpu.sync_copy(cur_hbm.at[pl.ds(pl.multiple_of(base, 8), C)], cur_buf)

    target = jnp.where(lane < 8, base, base + C).astype(_i32)

    def search_body(_, carry):
        lo, hi = carry
        mid = (lo + hi) >> 1
        pltpu.async_copy(ids_hbm.at[mid], probe_buf, sem_probe).wait()
        gathered = probe_buf[...]
        pred = gathered < target
        return jnp.where(pred, mid + 1, lo), jnp.where(pred, hi, mid)

    lo0 = jnp.zeros((L,), dtype=_i32)
    hi0 = jnp.full((L,), N_TICKS, dtype=_i32)
    lo_v, _ = lax.fori_loop(0, SEARCH_STEPS, search_body, (lo0, hi0))
    t0 = lo_v[0]
    t1 = lo_v[8]

    start = (t0 >> 3) << 3
    nblk = (t1 - start + (BLK - 1)) // BLK

    def blk_body(b, carry):
        off = jnp.minimum(start + b * BLK, N_TICKS - BLK)
        off = pl.multiple_of(off, 8)
        pltpu.async_copy(ids_hbm.at[pl.ds(off, BLK)], ids_buf, sem_blk).wait()
        pltpu.async_copy(prc_hbm.at[pl.ds(off, BLK)], prc_buf, sem_blk).wait()

        def vec_body(i, inner):
            voff = pl.multiple_of(i * L, L)
            idv = ids_buf[pl.ds(voff, L)]
            pv = prc_buf[pl.ds(voff, L)]
            loc = idv - base
            valid = (loc >= 0) & (loc < C)
            locc = jnp.where(valid, loc, 0)
            perm = (lane + i) & (L - 1)
            slot = perm * C + locc
            h = plsc.load_gather(acc_hi, [slot])
            lw = plsc.load_gather(acc_lo, [slot])
            plsc.store_scatter(acc_hi, [slot], jnp.maximum(h, pv), mask=valid)
            plsc.store_scatter(acc_lo, [slot], jnp.minimum(lw, pv), mask=valid)
            return inner

        lax.fori_loop(0, BLK // L, vec_body, 0)
        return carry

    lax.fori_loop(0, nblk, blk_body, 0)

    one = jnp.float32(1.0)
    zero = jnp.float32(0.0)

    def comb_body(j, carry):
        joff = j * L
        h = acc_hi[pl.ds(pl.multiple_of(joff, L), L)]
        lw = acc_lo[pl.ds(pl.multiple_of(joff, L), L)]
        for ln in range(1, L):
            h = jnp.maximum(h, acc_hi[pl.ds(pl.multiple_of(ln * C + joff, L), L)])
            lw = jnp.minimum(lw, acc_lo[pl.ds(pl.multiple_of(ln * C + joff, L), L)])
        empty = h == -jnp.inf
        h = jnp.where(empty, zero, h)
        lw = jnp.where(empty, zero, lw)
        cur = cur_buf[pl.ds(pl.multiple_of(joff, L), L)]
        sig = jnp.where(cur > h, one, jnp.where(cur < lw, -one, zero))
        sig_buf[pl.ds(pl.multiple_of(joff, L), L)] = sig
        return carry

    lax.fori_loop(0, C // L, comb_body, 0)
    pltpu.sync_copy(sig_buf, out_hbm.at[pl.ds(pl.multiple_of(base, 8), C)])


@jax.jit
def _run(inst_ids, tick_price, cur_price):
    mesh = plsc.VectorSubcoreMesh(core_axis_name="c", subcore_axis_name="s")
    kern = functools.partial(
        pl.kernel,
        mesh=mesh,
        compiler_params=pltpu.CompilerParams(needs_layout_passes=False),
        out_type=jax.ShapeDtypeStruct((NIDS,), _f32),
        scratch_types=[
            pltpu.VMEM((L * C,), _f32),
            pltpu.VMEM((L * C,), _f32),
            pltpu.VMEM((BLK,), _i32),
            pltpu.VMEM((BLK,), _f32),
            pltpu.VMEM((C,), _f32),
            pltpu.VMEM((C,), _f32),
            pltpu.VMEM((L,), _i32),
            pltpu.SemaphoreType.DMA,
            pltpu.SemaphoreType.DMA,
        ],
    )(_sc_body)
    cur_pad = jnp.concatenate(
        [cur_price, jnp.zeros((NIDS - NUM_INST,), dtype=_f32)])
    out = kern(inst_ids, tick_price, cur_pad)
    return out[:NUM_INST]


def kernel(timestamp, inst_ids, tick_price, cur_price):
    del timestamp
    return _run(inst_ids.astype(_i32), tick_price, cur_price)


# speedup vs baseline: 19.0425x; 1.0247x over previous
import functools

import jax
import jax.numpy as jnp
from jax import lax
from jax.experimental import pallas as pl
from jax.experimental.pallas import tpu as pltpu
from jax.experimental.pallas import tpu_sc as plsc

NUM_INST = 50000
N_TICKS = 3200000

NC = 2
NS = 16
L = 16
NW = NC * NS

C = 1568
NIDS = C * NW
BLK = 8192
SEARCH_STEPS = 22

_i32 = jnp.int32
_f32 = jnp.float32


def _sc_body(ids_hbm, prc_hbm, cur_hbm, out_hbm,
             acc_hi, acc_lo, ids_buf, prc_buf, cur_buf, sig_buf, probe_buf,
             sem_probe, sem_blk):
    lane = jnp.arange(L, dtype=_i32)
    w = lax.axis_index("s") * NC + lax.axis_index("c")
    base = w * C

    neg_inf = jnp.full((L,), -jnp.inf, dtype=_f32)
    pos_inf = jnp.full((L,), jnp.inf, dtype=_f32)

    def init_body(j, carry):
        off = pl.multiple_of(j * L, L)
        acc_hi[pl.ds(off, L)] = neg_inf
        acc_lo[pl.ds(off, L)] = pos_inf
        return carry

    lax.fori_loop(0, (L * C) // L, init_body, 0)

    pltpu.sync_copy(cur_hbm.at[pl.ds(pl.multiple_of(base, 8), C)], cur_buf)

    target = jnp.where(lane < 8, base, base + C).astype(_i32)

    def search_body(_, carry):
        lo, hi = carry
        mid = (lo + hi) >> 1
        pltpu.async_copy(ids_hbm.at[mid], probe_buf, sem_probe).wait()
        gathered = probe_buf[...]
        pred = gathered < target
        return jnp.where(pred, mid + 1, lo), jnp.where(pred, hi, mid)

    lo0 = jnp.zeros((L,), dtype=_i32)
    hi0 = jnp.full((L,), N_TICKS, dtype=_i32)
    lo_v, _ = lax.fori_loop(0, SEARCH_STEPS, search_body, (lo0, hi0))
    t0 = lo_v[0]
    t1 = lo_v[8]

    start = (t0 >> 3) << 3
    nblk = (t1 - start + (BLK - 1)) // BLK

    UNROLL = 4

    def blk_body(b, carry):
        off = jnp.minimum(start + b * BLK, N_TICKS - BLK)
        off = pl.multiple_of(off, 8)
        cp_ids = pltpu.async_copy(ids_hbm.at[pl.ds(off, BLK)], ids_buf, sem_blk)
        cp_prc = pltpu.async_copy(prc_hbm.at[pl.ds(off, BLK)], prc_buf, sem_blk)
        cp_ids.wait()
        cp_prc.wait()

        def vec_body(i, inner):
            for u in range(UNROLL):
                iu = i * UNROLL + u
                voff = pl.multiple_of(iu * L, L)
                idv = ids_buf[pl.ds(voff, L)]
                pv = prc_buf[pl.ds(voff, L)]
                loc = idv - base
                valid = (loc >= 0) & (loc < C)
                locc = jnp.where(valid, loc, 0)
                perm = (lane + iu) & (L - 1)
                slot = perm * C + locc
                h = plsc.load_gather(acc_hi, [slot])
                lw = plsc.load_gather(acc_lo, [slot])
                plsc.store_scatter(acc_hi, [slot], jnp.maximum(h, pv), mask=valid)
                plsc.store_scatter(acc_lo, [slot], jnp.minimum(lw, pv), mask=valid)
            return inner

        lax.fori_loop(0, BLK // (L * UNROLL), vec_body, 0)
        return carry

    lax.fori_loop(0, nblk, blk_body, 0)

    one = jnp.float32(1.0)
    zero = jnp.float32(0.0)

    def comb_body(j, carry):
        joff = j * L
        h = acc_hi[pl.ds(pl.multiple_of(joff, L), L)]
        lw = acc_lo[pl.ds(pl.multiple_of(joff, L), L)]
        for ln in range(1, L):
            h = jnp.maximum(h, acc_hi[pl.ds(pl.multiple_of(ln * C + joff, L), L)])
            lw = jnp.minimum(lw, acc_lo[pl.ds(pl.multiple_of(ln * C + joff, L), L)])
        empty = h == -jnp.inf
        h = jnp.where(empty, zero, h)
        lw = jnp.where(empty, zero, lw)
        cur = cur_buf[pl.ds(pl.multiple_of(joff, L), L)]
        sig = jnp.where(cur > h, one, jnp.where(cur < lw, -one, zero))
        sig_buf[pl.ds(pl.multiple_of(joff, L), L)] = sig
        return carry

    lax.fori_loop(0, C // L, comb_body, 0)
    pltpu.sync_copy(sig_buf, out_hbm.at[pl.ds(pl.multiple_of(base, 8), C)])


@jax.jit
def _run(inst_ids, tick_price, cur_price):
    mesh = plsc.VectorSubcoreMesh(core_axis_name="c", subcore_axis_name="s")
    kern = functools.partial(
        pl.kernel,
        mesh=mesh,
        compiler_params=pltpu.CompilerParams(needs_layout_passes=False),
        out_type=jax.ShapeDtypeStruct((NIDS,), _f32),
        scratch_types=[
            pltpu.VMEM((L * C,), _f32),
            pltpu.VMEM((L * C,), _f32),
            pltpu.VMEM((BLK,), _i32),
            pltpu.VMEM((BLK,), _f32),
            pltpu.VMEM((C,), _f32),
            pltpu.VMEM((C,), _f32),
            pltpu.VMEM((L,), _i32),
            pltpu.SemaphoreType.DMA,
            pltpu.SemaphoreType.DMA,
        ],
    )(_sc_body)
    cur_pad = jnp.concatenate(
        [cur_price, jnp.zeros((NIDS - NUM_INST,), dtype=_f32)])
    out = kern(inst_ids, tick_price, cur_pad)
    return out[:NUM_INST]


def kernel(timestamp, inst_ids, tick_price, cur_price):
    del timestamp
    return _run(inst_ids.astype(_i32), tick_price, cur_price)


# speedup vs baseline: 44.1143x; 2.3166x over previous
import functools

import jax
import jax.numpy as jnp
from jax import lax
from jax.experimental import pallas as pl
from jax.experimental.pallas import tpu as pltpu
from jax.experimental.pallas import tpu_sc as plsc

NUM_INST = 50000
N_TICKS = 3200000

NC = 2
NS = 16
L = 16
NW = NC * NS

C = 1568
NIDS = C * NW
BLK = 8192
SEARCH_STEPS = 22

_i32 = jnp.int32
_f32 = jnp.float32


def _sc_body(ids_hbm, prc_hbm, cur_hbm, out_hbm,
             acc_hi, acc_lo, ids_buf, prc_buf, cur_buf, sig_buf, probe_buf,
             sem_probe, sem_blk):
    lane = jnp.arange(L, dtype=_i32)
    w = lax.axis_index("s") * NC + lax.axis_index("c")
    base = w * C

    neg_inf = jnp.full((L,), -jnp.inf, dtype=_f32)
    pos_inf = jnp.full((L,), jnp.inf, dtype=_f32)

    def init_body(j, carry):
        off = pl.multiple_of(j * L, L)
        acc_hi[pl.ds(off, L)] = neg_inf
        acc_lo[pl.ds(off, L)] = pos_inf
        return carry

    lax.fori_loop(0, (L * C) // L, init_body, 0)

    pltpu.sync_copy(cur_hbm.at[pl.ds(pl.multiple_of(base, 8), C)], cur_buf)

    target = jnp.where(lane < 8, base, base + C).astype(_i32)

    def search_body(_, carry):
        lo, hi = carry
        mid = (lo + hi) >> 1
        pltpu.async_copy(ids_hbm.at[mid], probe_buf, sem_probe).wait()
        gathered = probe_buf[...]
        pred = gathered < target
        return jnp.where(pred, mid + 1, lo), jnp.where(pred, hi, mid)

    lo0 = jnp.zeros((L,), dtype=_i32)
    hi0 = jnp.full((L,), N_TICKS, dtype=_i32)
    lo_v, _ = lax.fori_loop(0, SEARCH_STEPS, search_body, (lo0, hi0))
    t0 = lo_v[0]
    t1 = lo_v[8]

    start = (t0 >> 3) << 3
    nblk = (t1 - start + (BLK - 1)) // BLK

    UNROLL = 4

    def blk_body(b, carry):
        off = jnp.minimum(start + b * BLK, N_TICKS - BLK)
        off = pl.multiple_of(off, 8)
        cp_ids = pltpu.async_copy(ids_hbm.at[pl.ds(off, BLK)], ids_buf, sem_blk)
        cp_prc = pltpu.async_copy(prc_hbm.at[pl.ds(off, BLK)], prc_buf, sem_blk)
        cp_ids.wait()
        cp_prc.wait()

        def vec_body(i, inner):
            for u in range(UNROLL):
                iu = i * UNROLL + u
                voff = pl.multiple_of(iu * L, L)
                idv = ids_buf[pl.ds(voff, L)]
                pv = prc_buf[pl.ds(voff, L)]
                loc = idv - base
                valid = (loc >= 0) & (loc < C)
                locc = jnp.where(valid, loc, 0)
                perm = (lane + iu) & (L - 1)
                slot = locc * L + perm
                h = plsc.load_gather(acc_hi, [slot])
                lw = plsc.load_gather(acc_lo, [slot])
                plsc.store_scatter(acc_hi, [slot], jnp.maximum(h, pv), mask=valid)
                plsc.store_scatter(acc_lo, [slot], jnp.minimum(lw, pv), mask=valid)
            return inner

        lax.fori_loop(0, BLK // (L * UNROLL), vec_body, 0)
        return carry

    lax.fori_loop(0, nblk, blk_body, 0)

    one = jnp.float32(1.0)
    zero = jnp.float32(0.0)

    def comb_body(j, carry):
        joff = j * L
        rowidx = (joff + lane) * L
        h = plsc.load_gather(acc_hi, [rowidx])
        lw = plsc.load_gather(acc_lo, [rowidx])
        for p in range(1, L):
            h = jnp.maximum(h, plsc.load_gather(acc_hi, [rowidx + p]))
            lw = jnp.minimum(lw, plsc.load_gather(acc_lo, [rowidx + p]))
        empty = h == -jnp.inf
        h = jnp.where(empty, zero, h)
        lw = jnp.where(empty, zero, lw)
        cur = cur_buf[pl.ds(pl.multiple_of(joff, L), L)]
        sig = jnp.where(cur > h, one, jnp.where(cur < lw, -one, zero))
        sig_buf[pl.ds(pl.multiple_of(joff, L), L)] = sig
        return carry

    lax.fori_loop(0, C // L, comb_body, 0)
    pltpu.sync_copy(sig_buf, out_hbm.at[pl.ds(pl.multiple_of(base, 8), C)])


@jax.jit
def _run(inst_ids, tick_price, cur_price):
    mesh = plsc.VectorSubcoreMesh(core_axis_name="c", subcore_axis_name="s")
    kern = functools.partial(
        pl.kernel,
        mesh=mesh,
        compiler_params=pltpu.CompilerParams(needs_layout_passes=False),
        out_type=jax.ShapeDtypeStruct((NIDS,), _f32),
        scratch_types=[
            pltpu.VMEM((L * C,), _f32),
            pltpu.VMEM((L * C,), _f32),
            pltpu.VMEM((BLK,), _i32),
            pltpu.VMEM((BLK,), _f32),
            pltpu.VMEM((C,), _f32),
            pltpu.VMEM((C,), _f32),
            pltpu.VMEM((L,), _i32),
            pltpu.SemaphoreType.DMA,
            pltpu.SemaphoreType.DMA,
        ],
    )(_sc_body)
    cur_pad = jnp.concatenate(
        [cur_price, jnp.zeros((NIDS - NUM_INST,), dtype=_f32)])
    out = kern(inst_ids, tick_price, cur_pad)
    return out[:NUM_INST]


def kernel(timestamp, inst_ids, tick_price, cur_price):
    del timestamp
    return _run(inst_ids.astype(_i32), tick_price, cur_price)
